# baseline (speedup 1.0000x reference)
import functools

import jax
import jax.numpy as jnp
from jax import lax
from jax.experimental import pallas as pl
from jax.experimental.pallas import tpu as pltpu
from jax.experimental.pallas import tpu_sc as plsc

L = 16


def _sc_add_kernel(B, S, D, R, n_workers):
    s_per_w = S // n_workers
    n_blocks = s_per_w // R
    assert n_blocks % 2 == 0 and S % n_workers == 0 and s_per_w % R == 0
    assert D % L == 0

    mesh = plsc.VectorSubcoreMesh(core_axis_name="c", subcore_axis_name="s")

    @functools.partial(
        pl.kernel,
        mesh=mesh,
        out_type=jax.ShapeDtypeStruct((B, S, D), jnp.float32),
        scratch_types=[
            pltpu.VMEM((R, D), jnp.float32),
            pltpu.VMEM((R, D), jnp.float32),
            pltpu.VMEM((B, R, D), jnp.float32),
            pltpu.VMEM((B, R, D), jnp.float32),
            pltpu.VMEM((B, R, D), jnp.float32),
            pltpu.VMEM((B, R, D), jnp.float32),
            pltpu.SemaphoreType.DMA,
            pltpu.SemaphoreType.DMA,
            pltpu.SemaphoreType.DMA((B,)),
            pltpu.SemaphoreType.DMA((B,)),
            pltpu.SemaphoreType.DMA,
            pltpu.SemaphoreType.DMA,
        ],
    )
    def k(x_hbm, p_hbm, o_hbm, pbuf_a, pbuf_b, xin_a, xin_b, xout_a, xout_b,
          semp_a, semp_b, semin_a, semin_b, semout_a, semout_b):
        nc = 2
        wid = lax.axis_index("s") * nc + lax.axis_index("c")
        base0 = wid * s_per_w

        def fire_in(blk, pbuf, xin, semp, semin):
            base = base0 + blk * R
            pltpu.async_copy(p_hbm.at[pl.ds(base, R)], pbuf, semp)
            for b in range(B):
                pltpu.async_copy(x_hbm.at[b, pl.ds(base, R)], xin.at[b], semin.at[b])

        def process(blk, pbuf, xin, xout, semp, semin, semout):
            base = base0 + blk * R
            pltpu.make_async_copy(p_hbm.at[pl.ds(base, R)], pbuf, semp).wait()
            for b in range(B):
                pltpu.make_async_copy(
                    x_hbm.at[b, pl.ds(base, R)], xin.at[b], semin.at[b]).wait()

            @pl.when(blk >= 2)
            def _():
                for b in range(B):
                    pltpu.make_async_copy(
                        xout.at[b], o_hbm.at[b, pl.ds(base, R)], semout).wait()

            @plsc.parallel_loop(0, D // L, unroll=2)
            def _(i):
                c = i * L
                for r in range(R):
                    pv = pbuf[r, pl.ds(c, L)]
                    for b in range(B):
                        xout[b, r, pl.ds(c, L)] = xin[b, r, pl.ds(c, L)] + pv

            for b in range(B):
                pltpu.async_copy(xout.at[b], o_hbm.at[b, pl.ds(base, R)], semout)

            @pl.when(blk + 2 < n_blocks)
            def _():
                fire_in(blk + 2, pbuf, xin, semp, semin)

        fire_in(0, pbuf_a, xin_a, semp_a, semin_a)
        fire_in(1, pbuf_b, xin_b, semp_b, semin_b)

        def pair_body(j, _):
            process(2 * j, pbuf_a, xin_a, xout_a, semp_a, semin_a, semout_a)
            process(2 * j + 1, pbuf_b, xin_b, xout_b, semp_b, semin_b, semout_b)
            return 0

        lax.fori_loop(0, n_blocks // 2, pair_body, 0)

        for blk, xout, semout in ((n_blocks - 2, xout_a, semout_a),
                                  (n_blocks - 1, xout_b, semout_b)):
            base = base0 + blk * R
            for b in range(B):
                pltpu.make_async_copy(
                    xout.at[b], o_hbm.at[b, pl.ds(base, R)], semout).wait()

    return k


def _tc_add(x, pos_table, b_lo, b_hi):
    B, S, D = x.shape
    BLK = 512

    def body(x_ref, p_ref, o_ref):
        o_ref[...] = x_ref[...] + p_ref[...][None]

    return pl.pallas_call(
        body,
        grid=(S // BLK, b_hi - b_lo),
        in_specs=[
            pl.BlockSpec((1, BLK, D), lambda i, b: (b + b_lo, i, 0)),
            pl.BlockSpec((BLK, D), lambda i, b: (i, 0)),
        ],
        out_specs=pl.BlockSpec((1, BLK, D), lambda i, b: (b, i, 0)),
        out_shape=jax.ShapeDtypeStruct((b_hi - b_lo, S, D), x.dtype),
    )(x, pos_table)


def kernel(x, pos_table):
    B, S, D = x.shape
    B_SC = 2
    k = _sc_add_kernel(B_SC, S, D, R=8, n_workers=32)
    out_sc = k(x, pos_table[:S])
    out_tc = _tc_add(x, pos_table[:S], B_SC, B)
    return jnp.concatenate([out_sc, out_tc], axis=0)


# speedup vs baseline: 1.5805x; 1.5805x over previous
import functools

import jax
import jax.numpy as jnp
from jax import lax
from jax.experimental import pallas as pl
from jax.experimental.pallas import tpu as pltpu
from jax.experimental.pallas import tpu_sc as plsc

L = 16


def _sc_add_kernel(B, S, D, R, n_workers):
    s_per_w = S // n_workers
    n_blocks = s_per_w // R
    assert n_blocks % 2 == 0 and S % n_workers == 0 and s_per_w % R == 0
    assert D % L == 0

    mesh = plsc.VectorSubcoreMesh(core_axis_name="c", subcore_axis_name="s")

    @functools.partial(
        pl.kernel,
        mesh=mesh,
        out_type=jax.ShapeDtypeStruct((B, S, D), jnp.float32),
        scratch_types=[
            pltpu.VMEM((R, D), jnp.float32),
            pltpu.VMEM((R, D), jnp.float32),
            pltpu.VMEM((B, R, D), jnp.float32),
            pltpu.VMEM((B, R, D), jnp.float32),
            pltpu.VMEM((B, R, D), jnp.float32),
            pltpu.VMEM((B, R, D), jnp.float32),
            pltpu.SemaphoreType.DMA,
            pltpu.SemaphoreType.DMA,
            pltpu.SemaphoreType.DMA((B,)),
            pltpu.SemaphoreType.DMA((B,)),
            pltpu.SemaphoreType.DMA,
            pltpu.SemaphoreType.DMA,
        ],
    )
    def k(x_hbm, p_hbm, o_hbm, pbuf_a, pbuf_b, xin_a, xin_b, xout_a, xout_b,
          semp_a, semp_b, semin_a, semin_b, semout_a, semout_b):
        nc = 2
        wid = lax.axis_index("s") * nc + lax.axis_index("c")
        base0 = wid * s_per_w

        def fire_in(blk, pbuf, xin, semp, semin):
            base = base0 + blk * R
            pltpu.async_copy(p_hbm.at[pl.ds(base, R)], pbuf, semp)
            for b in range(B):
                pltpu.async_copy(x_hbm.at[b, pl.ds(base, R)], xin.at[b], semin.at[b])

        def process(blk, pbuf, xin, xout, semp, semin, semout):
            base = base0 + blk * R
            pltpu.make_async_copy(p_hbm.at[pl.ds(base, R)], pbuf, semp).wait()
            for b in range(B):
                pltpu.make_async_copy(
                    x_hbm.at[b, pl.ds(base, R)], xin.at[b], semin.at[b]).wait()

            @pl.when(blk >= 2)
            def _():
                for b in range(B):
                    pltpu.make_async_copy(
                        xout.at[b], o_hbm.at[b, pl.ds(base, R)], semout).wait()

            @plsc.parallel_loop(0, D // L, unroll=4)
            def _(i):
                c = i * L
                for r in range(R):
                    pv = pbuf[r, pl.ds(c, L)]
                    for b in range(B):
                        xout[b, r, pl.ds(c, L)] = xin[b, r, pl.ds(c, L)] + pv

            for b in range(B):
                pltpu.async_copy(xout.at[b], o_hbm.at[b, pl.ds(base, R)], semout)

            @pl.when(blk + 2 < n_blocks)
            def _():
                fire_in(blk + 2, pbuf, xin, semp, semin)

        fire_in(0, pbuf_a, xin_a, semp_a, semin_a)
        fire_in(1, pbuf_b, xin_b, semp_b, semin_b)

        def pair_body(j, _):
            process(2 * j, pbuf_a, xin_a, xout_a, semp_a, semin_a, semout_a)
            process(2 * j + 1, pbuf_b, xin_b, xout_b, semp_b, semin_b, semout_b)
            return 0

        lax.fori_loop(0, n_blocks // 2, pair_body, 0)

        for blk, xout, semout in ((n_blocks - 2, xout_a, semout_a),
                                  (n_blocks - 1, xout_b, semout_b)):
            base = base0 + blk * R
            for b in range(B):
                pltpu.make_async_copy(
                    xout.at[b], o_hbm.at[b, pl.ds(base, R)], semout).wait()

    return k


def kernel(x, pos_table):
    B, S, D = x.shape
    k = _sc_add_kernel(B, S, D, R=8, n_workers=32)
    return k(x, pos_table[:S])


# speedup vs baseline: 1.5818x; 1.0008x over previous
import functools

import jax
import jax.numpy as jnp
from jax import lax
from jax.experimental import pallas as pl
from jax.experimental.pallas import tpu as pltpu
from jax.experimental.pallas import tpu_sc as plsc

L = 16
NSETS = 4


def _sc_add_kernel(B, S, D, R, n_workers):
    s_per_w = S // n_workers
    n_blocks = s_per_w // R
    assert n_blocks % NSETS == 0 and S % n_workers == 0 and s_per_w % R == 0
    assert D % L == 0 and n_blocks >= NSETS

    mesh = plsc.VectorSubcoreMesh(core_axis_name="c", subcore_axis_name="s")

    scratch = []
    for _ in range(NSETS):
        scratch.append(pltpu.VMEM((R, D), jnp.float32))
        scratch.append(pltpu.VMEM((B, R, D), jnp.float32))
        scratch.append(pltpu.SemaphoreType.DMA)
        scratch.append(pltpu.SemaphoreType.DMA((B,)))
        scratch.append(pltpu.SemaphoreType.DMA)

    @functools.partial(
        pl.kernel,
        mesh=mesh,
        out_type=jax.ShapeDtypeStruct((B, S, D), jnp.float32),
        scratch_types=scratch,
    )
    def k(x_hbm, p_hbm, o_hbm, *bufs):
        sets = [bufs[5 * i: 5 * i + 5] for i in range(NSETS)]
        nc = 2
        wid = lax.axis_index("s") * nc + lax.axis_index("c")
        base0 = wid * s_per_w

        def fire_in(blk, st):
            pbuf, xbuf, semp, semin, _ = st
            base = base0 + blk * R
            pltpu.async_copy(p_hbm.at[pl.ds(base, R)], pbuf, semp)
            for b in range(B):
                pltpu.async_copy(x_hbm.at[b, pl.ds(base, R)], xbuf.at[b], semin.at[b])

        def drain_out(blk, st):
            _, xbuf, _, _, semout = st
            base = base0 + blk * R
            for b in range(B):
                pltpu.make_async_copy(
                    xbuf.at[b], o_hbm.at[b, pl.ds(base, R)], semout).wait()

        def process(blk, st, st_ahead):
            pbuf, xbuf, semp, semin, semout = st
            base = base0 + blk * R

            @pl.when(blk >= 2)
            def _():
                drain_out(blk - 2, st_ahead)

            @pl.when(blk + 2 < n_blocks)
            def _():
                fire_in(blk + 2, st_ahead)

            pltpu.make_async_copy(p_hbm.at[pl.ds(base, R)], pbuf, semp).wait()
            for b in range(B):
                pltpu.make_async_copy(
                    x_hbm.at[b, pl.ds(base, R)], xbuf.at[b], semin.at[b]).wait()

            @plsc.parallel_loop(0, D // L, unroll=2)
            def _(i):
                c = i * L
                for r in range(R):
                    pv = pbuf[r, pl.ds(c, L)]
                    for b in range(B):
                        xbuf[b, r, pl.ds(c, L)] = xbuf[b, r, pl.ds(c, L)] + pv

            for b in range(B):
                pltpu.async_copy(xbuf.at[b], o_hbm.at[b, pl.ds(base, R)], semout)

        fire_in(0, sets[0])
        fire_in(1, sets[1])

        def quad_body(j, _):
            blk0 = NSETS * j
            for q in range(NSETS):
                process(blk0 + q, sets[q], sets[(q + 2) % NSETS])
            return 0

        lax.fori_loop(0, n_blocks // NSETS, quad_body, 0)

        drain_out(n_blocks - 2, sets[(n_blocks - 2) % NSETS])
        drain_out(n_blocks - 1, sets[(n_blocks - 1) % NSETS])

    return k


def kernel(x, pos_table):
    B, S, D = x.shape
    k = _sc_add_kernel(B, S, D, R=8, n_workers=32)
    return k(x, pos_table[:S])


# speedup vs baseline: 1.5931x; 1.0071x over previous
import functools

import jax
import jax.numpy as jnp
from jax import lax
from jax.experimental import pallas as pl
from jax.experimental.pallas import tpu as pltpu
from jax.experimental.pallas import tpu_sc as plsc

L = 16


def _sc_add_kernel(B, S, D, R, n_workers):
    s_per_w = S // n_workers
    n_blocks = s_per_w // R
    assert n_blocks % 2 == 0 and S % n_workers == 0 and s_per_w % R == 0
    assert D % L == 0

    mesh = plsc.VectorSubcoreMesh(core_axis_name="c", subcore_axis_name="s")

    @functools.partial(
        pl.kernel,
        mesh=mesh,
        out_type=jax.ShapeDtypeStruct((B, S, D), jnp.float32),
        scratch_types=[
            pltpu.VMEM((R, D), jnp.float32),
            pltpu.VMEM((R, D), jnp.float32),
            pltpu.VMEM((B, R, D), jnp.float32),
            pltpu.VMEM((B, R, D), jnp.float32),
            pltpu.VMEM((B, R, D), jnp.float32),
            pltpu.VMEM((B, R, D), jnp.float32),
            pltpu.SemaphoreType.DMA,
            pltpu.SemaphoreType.DMA,
            pltpu.SemaphoreType.DMA((B,)),
            pltpu.SemaphoreType.DMA((B,)),
            pltpu.SemaphoreType.DMA,
            pltpu.SemaphoreType.DMA,
        ],
    )
    def k(x_hbm, p_hbm, o_hbm, pbuf_a, pbuf_b, xin_a, xin_b, xout_a, xout_b,
          semp_a, semp_b, semin_a, semin_b, semout_a, semout_b):
        nc = 2
        wid = lax.axis_index("s") * nc + lax.axis_index("c")
        base0 = wid * s_per_w

        def fire_in(blk, pbuf, xin, semp, semin):
            base = base0 + blk * R
            pltpu.async_copy(p_hbm.at[pl.ds(base, R)], pbuf, semp)
            for b in range(B):
                pltpu.async_copy(x_hbm.at[b, pl.ds(base, R)], xin.at[b], semin.at[b])

        def process(blk, pbuf, xin, xout, semp, semin, semout):
            base = base0 + blk * R
            pltpu.make_async_copy(p_hbm.at[pl.ds(base, R)], pbuf, semp).wait()
            for b in range(B):
                pltpu.make_async_copy(
                    x_hbm.at[b, pl.ds(base, R)], xin.at[b], semin.at[b]).wait()

            @pl.when(blk >= 2)
            def _():
                for b in range(B):
                    pltpu.make_async_copy(
                        xout.at[b], o_hbm.at[b, pl.ds(base, R)], semout).wait()

            @plsc.parallel_loop(0, D // L, unroll=2)
            def _(i):
                c = i * L
                for r in range(R):
                    pv = pbuf[r, pl.ds(c, L)]
                    for b in range(B):
                        xout[b, r, pl.ds(c, L)] = xin[b, r, pl.ds(c, L)] + pv

            for b in range(B):
                pltpu.async_copy(xout.at[b], o_hbm.at[b, pl.ds(base, R)], semout)

            @pl.when(blk + 2 < n_blocks)
            def _():
                fire_in(blk + 2, pbuf, xin, semp, semin)

        fire_in(0, pbuf_a, xin_a, semp_a, semin_a)
        fire_in(1, pbuf_b, xin_b, semp_b, semin_b)

        def pair_body(j, _):
            process(2 * j, pbuf_a, xin_a, xout_a, semp_a, semin_a, semout_a)
            process(2 * j + 1, pbuf_b, xin_b, xout_b, semp_b, semin_b, semout_b)
            return 0

        lax.fori_loop(0, n_blocks // 2, pair_body, 0)

        for blk, xout, semout in ((n_blocks - 2, xout_a, semout_a),
                                  (n_blocks - 1, xout_b, semout_b)):
            base = base0 + blk * R
            for b in range(B):
                pltpu.make_async_copy(
                    xout.at[b], o_hbm.at[b, pl.ds(base, R)], semout).wait()

    return k


def kernel(x, pos_table):
    B, S, D = x.shape
    k = _sc_add_kernel(B, S, D, R=4, n_workers=32)
    return k(x, pos_table[:S])


# speedup vs baseline: 1.6130x; 1.0125x over previous
import functools

import jax
import jax.numpy as jnp
from jax import lax
from jax.experimental import pallas as pl
from jax.experimental.pallas import tpu as pltpu
from jax.experimental.pallas import tpu_sc as plsc

L = 16


def _sc_add_kernel(B, S, D, R, n_workers):
    s_per_w = S // n_workers
    n_blocks = s_per_w // R
    assert n_blocks % 2 == 0 and S % n_workers == 0 and s_per_w % R == 0
    assert D % L == 0

    mesh = plsc.VectorSubcoreMesh(core_axis_name="c", subcore_axis_name="s")

    @functools.partial(
        pl.kernel,
        mesh=mesh,
        out_type=jax.ShapeDtypeStruct((B, S, D), jnp.float32),
        scratch_types=[
            pltpu.VMEM((R, D), jnp.float32),
            pltpu.VMEM((R, D), jnp.float32),
            pltpu.VMEM((B, R, D), jnp.float32),
            pltpu.VMEM((B, R, D), jnp.float32),
            pltpu.VMEM((B, R, D), jnp.float32),
            pltpu.VMEM((B, R, D), jnp.float32),
            pltpu.SemaphoreType.DMA,
            pltpu.SemaphoreType.DMA,
            pltpu.SemaphoreType.DMA,
            pltpu.SemaphoreType.DMA,
            pltpu.SemaphoreType.DMA,
            pltpu.SemaphoreType.DMA,
        ],
    )
    def k(x_hbm, p_hbm, o_hbm, pbuf_a, pbuf_b, xin_a, xin_b, xout_a, xout_b,
          semp_a, semp_b, semin_a, semin_b, semout_a, semout_b):
        nc = 2
        wid = lax.axis_index("s") * nc + lax.axis_index("c")
        base0 = wid * s_per_w

        def fire_in(blk, pbuf, xin, semp, semin):
            base = base0 + blk * R
            pltpu.async_copy(p_hbm.at[pl.ds(base, R)], pbuf, semp)
            pltpu.async_copy(x_hbm.at[:, pl.ds(base, R)], xin, semin)

        def process(blk, pbuf, xin, xout, semp, semin, semout):
            base = base0 + blk * R
            pltpu.make_async_copy(p_hbm.at[pl.ds(base, R)], pbuf, semp).wait()
            pltpu.make_async_copy(x_hbm.at[:, pl.ds(base, R)], xin, semin).wait()

            @pl.when(blk >= 2)
            def _():
                pltpu.make_async_copy(
                    xout, o_hbm.at[:, pl.ds(base, R)], semout).wait()

            @plsc.parallel_loop(0, D // L, unroll=2)
            def _(i):
                c = i * L
                for r in range(R):
                    pv = pbuf[r, pl.ds(c, L)]
                    for b in range(B):
                        xout[b, r, pl.ds(c, L)] = xin[b, r, pl.ds(c, L)] + pv

            pltpu.async_copy(xout, o_hbm.at[:, pl.ds(base, R)], semout)

            @pl.when(blk + 2 < n_blocks)
            def _():
                fire_in(blk + 2, pbuf, xin, semp, semin)

        fire_in(0, pbuf_a, xin_a, semp_a, semin_a)
        fire_in(1, pbuf_b, xin_b, semp_b, semin_b)

        def pair_body(j, _):
            process(2 * j, pbuf_a, xin_a, xout_a, semp_a, semin_a, semout_a)
            process(2 * j + 1, pbuf_b, xin_b, xout_b, semp_b, semin_b, semout_b)
            return 0

        lax.fori_loop(0, n_blocks // 2, pair_body, 0)

        for blk, xout, semout in ((n_blocks - 2, xout_a, semout_a),
                                  (n_blocks - 1, xout_b, semout_b)):
            base = base0 + blk * R
            pltpu.make_async_copy(xout, o_hbm.at[:, pl.ds(base, R)], semout).wait()

    return k


def kernel(x, pos_table):
    B, S, D = x.shape
    k = _sc_add_kernel(B, S, D, R=8, n_workers=32)
    return k(x, pos_table[:S])


# speedup vs baseline: 1.6162x; 1.0020x over previous
import functools

import jax
import jax.numpy as jnp
from jax import lax
from jax.experimental import pallas as pl
from jax.experimental.pallas import tpu as pltpu
from jax.experimental.pallas import tpu_sc as plsc

L = 16


def _sc_add_kernel(B, S, D, R, n_workers):
    s_per_w = S // n_workers
    n_blocks = s_per_w // R
    assert n_blocks % 2 == 0 and S % n_workers == 0 and s_per_w % R == 0
    assert D % L == 0

    mesh = plsc.VectorSubcoreMesh(core_axis_name="c", subcore_axis_name="s")

    @functools.partial(
        pl.kernel,
        mesh=mesh,
        out_type=jax.ShapeDtypeStruct((B, S, D), jnp.float32),
        scratch_types=[
            pltpu.VMEM((R, D), jnp.float32),
            pltpu.VMEM((R, D), jnp.float32),
            pltpu.VMEM((B, R, D), jnp.float32),
            pltpu.VMEM((B, R, D), jnp.float32),
            pltpu.VMEM((B, R, D), jnp.float32),
            pltpu.VMEM((B, R, D), jnp.float32),
            pltpu.SemaphoreType.DMA,
            pltpu.SemaphoreType.DMA,
            pltpu.SemaphoreType.DMA,
            pltpu.SemaphoreType.DMA,
            pltpu.SemaphoreType.DMA,
            pltpu.SemaphoreType.DMA,
        ],
    )
    def k(x_hbm, p_hbm, o_hbm, pbuf_a, pbuf_b, xin_a, xin_b, xout_a, xout_b,
          semp_a, semp_b, semin_a, semin_b, semout_a, semout_b):
        ns = 16
        wid = lax.axis_index("c") * ns + lax.axis_index("s")
        base0 = wid * s_per_w

        def fire_in(blk, pbuf, xin, semp, semin):
            base = base0 + blk * R
            pltpu.async_copy(p_hbm.at[pl.ds(base, R)], pbuf, semp)
            pltpu.async_copy(x_hbm.at[:, pl.ds(base, R)], xin, semin)

        def process(blk, pbuf, xin, xout, semp, semin, semout):
            base = base0 + blk * R
            pltpu.make_async_copy(p_hbm.at[pl.ds(base, R)], pbuf, semp).wait()
            pltpu.make_async_copy(x_hbm.at[:, pl.ds(base, R)], xin, semin).wait()

            @pl.when(blk >= 2)
            def _():
                pltpu.make_async_copy(
                    xout, o_hbm.at[:, pl.ds(base, R)], semout).wait()

            @plsc.parallel_loop(0, D // L, unroll=2)
            def _(i):
                c = i * L
                for r in range(R):
                    pv = pbuf[r, pl.ds(c, L)]
                    for b in range(B):
                        xout[b, r, pl.ds(c, L)] = xin[b, r, pl.ds(c, L)] + pv

            pltpu.async_copy(xout, o_hbm.at[:, pl.ds(base, R)], semout)

            @pl.when(blk + 2 < n_blocks)
            def _():
                fire_in(blk + 2, pbuf, xin, semp, semin)

        fire_in(0, pbuf_a, xin_a, semp_a, semin_a)
        fire_in(1, pbuf_b, xin_b, semp_b, semin_b)

        def pair_body(j, _):
            process(2 * j, pbuf_a, xin_a, xout_a, semp_a, semin_a, semout_a)
            process(2 * j + 1, pbuf_b, xin_b, xout_b, semp_b, semin_b, semout_b)
            return 0

        lax.fori_loop(0, n_blocks // 2, pair_body, 0)

        for blk, xout, semout in ((n_blocks - 2, xout_a, semout_a),
                                  (n_blocks - 1, xout_b, semout_b)):
            base = base0 + blk * R
            pltpu.make_async_copy(xout, o_hbm.at[:, pl.ds(base, R)], semout).wait()

    return k


def kernel(x, pos_table):
    B, S, D = x.shape
    k = _sc_add_kernel(B, S, D, R=8, n_workers=32)
    return k(x, pos_table[:S])
